# baseline (speedup 1.0000x reference)
import jax
import jax.numpy as jnp
from jax import lax
from jax.experimental import pallas as pl
from jax.experimental.pallas import tpu as pltpu
from jax.experimental.pallas import tpu_sc as plsc

DIM = 128
N_EDGES = 320000
N_NODES = 10000
EDGE_NUM = 3000
NUM_CORES = 2
NUM_SUBCORES = 16
NW = NUM_CORES * NUM_SUBCORES
E_PER_W = N_EDGES // NW
LANES = 16
CHUNK = 32
N_FULL = E_PER_W // CHUNK
TAIL = E_PER_W - N_FULL * CHUNK
NBUF = 4


EIW = 10112


def _sc_body(nt_hbm, ei_hbm, table_hbm, out_hbm,
             nt_v, ei_v, et_v, rows_v, tab_sh, isem, gsem, osem):
    sid = lax.axis_index("s")
    wid = sid * NUM_CORES + lax.axis_index("c")
    base = wid * E_PER_W
    off = lax.rem(base, 128)
    cbase = pl.multiple_of(base - off, 128)

    @pl.when(sid < 7)
    def _stage():
        sl = pl.ds(sid * 384, 384)
        pltpu.sync_copy(table_hbm.at[sl], tab_sh.at[sl])

    @pl.when(sid == 7)
    def _stage_tail():
        sl = pl.ds(2688, 312)
        pltpu.sync_copy(table_hbm.at[sl], tab_sh.at[sl])

    pltpu.async_copy(nt_hbm, nt_v, isem)
    pltpu.async_copy(ei_hbm.at[:, pl.ds(cbase, EIW)], ei_v, isem)
    pltpu.make_async_copy(nt_hbm, nt_v, isem).wait()
    pltpu.make_async_copy(ei_hbm.at[:, pl.ds(cbase, EIW)], ei_v, isem).wait()

    plsc.subcore_barrier()

    def compute_span(lo, n):
        @plsc.parallel_loop(0, n, LANES, unroll=n // LANES)
        def _body(i):
            sl = pl.ds(off + lo + i, LANES)
            a = plsc.load_gather(nt_v, [ei_v[0, sl]])
            b = plsc.load_gather(nt_v, [ei_v[1, sl]])
            s = a + b
            et_v[pl.ds(lo + i, LANES)] = (
                lax.shift_right_logical(s * (s + 1), 1) + b)

    def start_gather(g, b):
        idx = et_v.at[pl.ds(g * CHUNK, CHUNK)]
        pltpu.async_copy(tab_sh.at[idx], rows_v.at[b], gsem.at[b])

    def wait_gather(g, b):
        idx = et_v.at[pl.ds(g * CHUNK, CHUNK)]
        pltpu.make_async_copy(tab_sh.at[idx], rows_v.at[b], gsem.at[b]).wait()

    def start_out(g, b):
        pltpu.async_copy(rows_v.at[b], out_hbm.at[pl.ds(base + g * CHUNK, CHUNK)],
                         osem.at[b])

    def wait_out(g, b):
        pltpu.make_async_copy(rows_v.at[b],
                              out_hbm.at[pl.ds(base + g * CHUNK, CHUNK)],
                              osem.at[b]).wait()

    compute_span(0, CHUNK)
    start_gather(0, 0)
    compute_span(CHUNK, CHUNK)
    start_gather(1, 1)
    wait_gather(0, 0)
    start_out(0, 0)
    compute_span(2 * CHUNK, CHUNK)
    start_gather(2, 2)
    wait_gather(1, 1)
    start_out(1, 1)
    compute_span(3 * CHUNK, CHUNK)
    start_gather(3, 3)

    def copy_chunk(g, carry):
        b = lax.rem(g, NBUF)
        wait_gather(g, b)
        start_out(g, b)
        compute_span((g + 2) * CHUNK, CHUNK)
        wait_out(g - 2, lax.rem(g + NBUF - 2, NBUF))
        start_gather(g + 2, lax.rem(g + 2, NBUF))
        return carry

    lax.fori_loop(2, N_FULL - 2, copy_chunk, 0)

    t_idx = et_v.at[pl.ds(N_FULL * CHUNK, TAIL)]
    t_rows = rows_v.at[0, pl.ds(0, TAIL)]
    t_out = out_hbm.at[pl.ds(base + N_FULL * CHUNK, TAIL)]

    g = N_FULL - 2
    wait_gather(g, g % NBUF)
    start_out(g, g % NBUF)
    compute_span(N_FULL * CHUNK, TAIL)
    wait_out(g - 2, (g - 2) % NBUF)
    g = N_FULL - 1
    wait_gather(g, g % NBUF)
    start_out(g, g % NBUF)
    wait_out(g - 2, (g - 2) % NBUF)
    wait_out(N_FULL - 2, (N_FULL - 2) % NBUF)
    pltpu.async_copy(tab_sh.at[t_idx], t_rows, gsem.at[0])
    pltpu.make_async_copy(tab_sh.at[t_idx], t_rows, gsem.at[0]).wait()
    pltpu.async_copy(t_rows, t_out, osem.at[0])
    wait_out(N_FULL - 1, (N_FULL - 1) % NBUF)
    pltpu.make_async_copy(t_rows, t_out, osem.at[0]).wait()


def kernel(node_type, edge_index, table):
    mesh = plsc.VectorSubcoreMesh(core_axis_name="c", subcore_axis_name="s")
    k = pl.kernel(
        _sc_body,
        mesh=mesh,
        out_type=jax.ShapeDtypeStruct((N_EDGES, DIM), jnp.float32),
        compiler_params=pltpu.CompilerParams(needs_layout_passes=False),
        scratch_types=[
            pltpu.VMEM((N_NODES,), jnp.int32),
            pltpu.VMEM((2, EIW), jnp.int32),
            pltpu.VMEM((E_PER_W,), jnp.int32),
            pltpu.VMEM((NBUF, CHUNK, DIM), jnp.float32),
            pltpu.VMEM_SHARED((EDGE_NUM, DIM), jnp.float32),
            pltpu.SemaphoreType.DMA,
            pltpu.SemaphoreType.DMA((NBUF,)),
            pltpu.SemaphoreType.DMA((NBUF,)),
        ],
    )
    return k(node_type, edge_index, table)


# speedup vs baseline: 1.0295x; 1.0295x over previous
import jax
import jax.numpy as jnp
from jax import lax
from jax.experimental import pallas as pl
from jax.experimental.pallas import tpu as pltpu
from jax.experimental.pallas import tpu_sc as plsc

DIM = 128
N_EDGES = 320000
N_NODES = 10000
EDGE_NUM = 3000
NUM_CORES = 2
NUM_SUBCORES = 16
NW = NUM_CORES * NUM_SUBCORES
E_PER_W = N_EDGES // NW
LANES = 16
CHUNK = 64
N_FULL = E_PER_W // CHUNK
TAIL = E_PER_W - N_FULL * CHUNK
NBUF = 6


EIW = 10112


def _sc_body(nt_hbm, ei_hbm, table_hbm, out_hbm,
             nt_v, ei_v, et_v, rows_v, tab_sh, isem, gsem, osem):
    sid = lax.axis_index("s")
    wid = sid * NUM_CORES + lax.axis_index("c")
    base = wid * E_PER_W
    off = lax.rem(base, 128)
    cbase = pl.multiple_of(base - off, 128)

    @pl.when(sid < 7)
    def _stage():
        sl = pl.ds(sid * 384, 384)
        pltpu.sync_copy(table_hbm.at[sl], tab_sh.at[sl])

    @pl.when(sid == 7)
    def _stage_tail():
        sl = pl.ds(2688, 312)
        pltpu.sync_copy(table_hbm.at[sl], tab_sh.at[sl])

    pltpu.async_copy(nt_hbm, nt_v, isem)
    pltpu.async_copy(ei_hbm.at[:, pl.ds(cbase, EIW)], ei_v, isem)
    pltpu.make_async_copy(nt_hbm, nt_v, isem).wait()
    pltpu.make_async_copy(ei_hbm.at[:, pl.ds(cbase, EIW)], ei_v, isem).wait()

    plsc.subcore_barrier()

    def compute_span(lo, n):
        @plsc.parallel_loop(0, n, LANES, unroll=n // LANES)
        def _body(i):
            sl = pl.ds(off + lo + i, LANES)
            a = plsc.load_gather(nt_v, [ei_v[0, sl]])
            b = plsc.load_gather(nt_v, [ei_v[1, sl]])
            s = a + b
            et_v[pl.ds(lo + i, LANES)] = (
                lax.shift_right_logical(s * (s + 1), 1) + b)

    def start_gather(g, b):
        idx = et_v.at[pl.ds(g * CHUNK, CHUNK)]
        pltpu.async_copy(tab_sh.at[idx], rows_v.at[b], gsem.at[b])

    def wait_gather(g, b):
        idx = et_v.at[pl.ds(g * CHUNK, CHUNK)]
        pltpu.make_async_copy(tab_sh.at[idx], rows_v.at[b], gsem.at[b]).wait()

    def start_out(g, b):
        pltpu.async_copy(rows_v.at[b], out_hbm.at[pl.ds(base + g * CHUNK, CHUNK)],
                         osem.at[b])

    def wait_out(g, b):
        pltpu.make_async_copy(rows_v.at[b],
                              out_hbm.at[pl.ds(base + g * CHUNK, CHUNK)],
                              osem.at[b]).wait()

    compute_span(0, CHUNK)
    start_gather(0, 0)
    compute_span(CHUNK, CHUNK)
    start_gather(1, 1)
    for p in range(4):
        wait_gather(p, p)
        start_out(p, p)
        compute_span((p + 2) * CHUNK, CHUNK)
        start_gather(p + 2, p + 2)

    def copy_chunk(g, carry):
        b = lax.rem(g, NBUF)
        wait_gather(g, b)
        start_out(g, b)
        compute_span((g + 2) * CHUNK, CHUNK)
        wait_out(g - 4, lax.rem(g + NBUF - 4, NBUF))
        start_gather(g + 2, lax.rem(g + 2, NBUF))
        return carry

    lax.fori_loop(4, N_FULL - 2, copy_chunk, 0)

    t_idx = et_v.at[pl.ds(N_FULL * CHUNK, TAIL)]
    t_rows = rows_v.at[0, pl.ds(0, TAIL)]
    t_out = out_hbm.at[pl.ds(base + N_FULL * CHUNK, TAIL)]

    g = N_FULL - 2
    wait_gather(g, g % NBUF)
    start_out(g, g % NBUF)
    compute_span(N_FULL * CHUNK, TAIL)
    wait_out(g - 4, (g - 4) % NBUF)
    g = N_FULL - 1
    wait_gather(g, g % NBUF)
    start_out(g, g % NBUF)
    wait_out(g - 4, (g - 4) % NBUF)
    wait_out(N_FULL - 4, (N_FULL - 4) % NBUF)
    wait_out(N_FULL - 3, (N_FULL - 3) % NBUF)
    wait_out(N_FULL - 2, (N_FULL - 2) % NBUF)
    pltpu.async_copy(tab_sh.at[t_idx], t_rows, gsem.at[0])
    pltpu.make_async_copy(tab_sh.at[t_idx], t_rows, gsem.at[0]).wait()
    pltpu.async_copy(t_rows, t_out, osem.at[0])
    wait_out(N_FULL - 1, (N_FULL - 1) % NBUF)
    pltpu.make_async_copy(t_rows, t_out, osem.at[0]).wait()


def kernel(node_type, edge_index, table):
    mesh = plsc.VectorSubcoreMesh(core_axis_name="c", subcore_axis_name="s")
    k = pl.kernel(
        _sc_body,
        mesh=mesh,
        out_type=jax.ShapeDtypeStruct((N_EDGES, DIM), jnp.float32),
        compiler_params=pltpu.CompilerParams(needs_layout_passes=False),
        scratch_types=[
            pltpu.VMEM((N_NODES,), jnp.int32),
            pltpu.VMEM((2, EIW), jnp.int32),
            pltpu.VMEM((E_PER_W,), jnp.int32),
            pltpu.VMEM((NBUF, CHUNK, DIM), jnp.float32),
            pltpu.VMEM_SHARED((EDGE_NUM, DIM), jnp.float32),
            pltpu.SemaphoreType.DMA,
            pltpu.SemaphoreType.DMA((NBUF,)),
            pltpu.SemaphoreType.DMA((NBUF,)),
        ],
    )
    return k(node_type, edge_index, table)


# speedup vs baseline: 1.0353x; 1.0057x over previous
import jax
import jax.numpy as jnp
from jax import lax
from jax.experimental import pallas as pl
from jax.experimental.pallas import tpu as pltpu
from jax.experimental.pallas import tpu_sc as plsc

DIM = 128
N_EDGES = 320000
N_NODES = 10000
EDGE_NUM = 3000
NUM_CORES = 2
NUM_SUBCORES = 16
NW = NUM_CORES * NUM_SUBCORES
E_PER_W = N_EDGES // NW
LANES = 16
CHUNK = 64
N_FULL = E_PER_W // CHUNK
TAIL = E_PER_W - N_FULL * CHUNK
NBUF = 4


EIW = 10112


def _sc_body(nt_hbm, ei_hbm, table_hbm, out_hbm,
             nt_v, ei_v, et_v, rows_v, tab_sh, isem, gsem, osem):
    sid = lax.axis_index("s")
    wid = sid * NUM_CORES + lax.axis_index("c")
    base = wid * E_PER_W
    off = lax.rem(base, 128)
    cbase = pl.multiple_of(base - off, 128)

    @pl.when(sid < 7)
    def _stage():
        sl = pl.ds(sid * 384, 384)
        pltpu.sync_copy(table_hbm.at[sl], tab_sh.at[sl])

    @pl.when(sid == 7)
    def _stage_tail():
        sl = pl.ds(2688, 312)
        pltpu.sync_copy(table_hbm.at[sl], tab_sh.at[sl])

    pltpu.async_copy(nt_hbm, nt_v, isem)
    pltpu.async_copy(ei_hbm.at[:, pl.ds(cbase, EIW)], ei_v, isem)
    pltpu.make_async_copy(nt_hbm, nt_v, isem).wait()
    pltpu.make_async_copy(ei_hbm.at[:, pl.ds(cbase, EIW)], ei_v, isem).wait()

    plsc.subcore_barrier()

    def compute_span(lo, n):
        @plsc.parallel_loop(0, n, LANES, unroll=n // LANES)
        def _body(i):
            sl = pl.ds(off + lo + i, LANES)
            a = plsc.load_gather(nt_v, [ei_v[0, sl]])
            b = plsc.load_gather(nt_v, [ei_v[1, sl]])
            s = a + b
            et_v[pl.ds(lo + i, LANES)] = (
                lax.shift_right_logical(s * (s + 1), 1) + b)

    def start_gather(g, b):
        idx = et_v.at[pl.ds(g * CHUNK, CHUNK)]
        pltpu.async_copy(tab_sh.at[idx], rows_v.at[b], gsem.at[b])

    def wait_gather(g, b):
        idx = et_v.at[pl.ds(g * CHUNK, CHUNK)]
        pltpu.make_async_copy(tab_sh.at[idx], rows_v.at[b], gsem.at[b]).wait()

    def start_out(g, b):
        pltpu.async_copy(rows_v.at[b], out_hbm.at[pl.ds(base + g * CHUNK, CHUNK)],
                         osem.at[b])

    def wait_out(g, b):
        pltpu.make_async_copy(rows_v.at[b],
                              out_hbm.at[pl.ds(base + g * CHUNK, CHUNK)],
                              osem.at[b]).wait()

    compute_span(0, CHUNK)
    start_gather(0, 0)
    compute_span(CHUNK, CHUNK)
    start_gather(1, 1)
    wait_gather(0, 0)
    start_out(0, 0)
    compute_span(2 * CHUNK, CHUNK)
    start_gather(2, 2)
    wait_gather(1, 1)
    start_out(1, 1)
    compute_span(3 * CHUNK, CHUNK)
    start_gather(3, 3)

    def copy_chunk(g, carry):
        b = lax.rem(g, NBUF)
        wait_gather(g, b)
        start_out(g, b)
        compute_span((g + 2) * CHUNK, CHUNK)
        wait_out(g - 2, lax.rem(g + NBUF - 2, NBUF))
        start_gather(g + 2, lax.rem(g + 2, NBUF))
        return carry

    lax.fori_loop(2, N_FULL - 2, copy_chunk, 0)

    t_idx = et_v.at[pl.ds(N_FULL * CHUNK, TAIL)]
    t_rows = rows_v.at[0, pl.ds(0, TAIL)]
    t_out = out_hbm.at[pl.ds(base + N_FULL * CHUNK, TAIL)]

    g = N_FULL - 2
    wait_gather(g, g % NBUF)
    start_out(g, g % NBUF)
    compute_span(N_FULL * CHUNK, TAIL)
    wait_out(g - 2, (g - 2) % NBUF)
    g = N_FULL - 1
    wait_gather(g, g % NBUF)
    start_out(g, g % NBUF)
    wait_out(g - 2, (g - 2) % NBUF)
    wait_out(N_FULL - 2, (N_FULL - 2) % NBUF)
    pltpu.async_copy(tab_sh.at[t_idx], t_rows, gsem.at[0])
    pltpu.make_async_copy(tab_sh.at[t_idx], t_rows, gsem.at[0]).wait()
    pltpu.async_copy(t_rows, t_out, osem.at[0])
    wait_out(N_FULL - 1, (N_FULL - 1) % NBUF)
    pltpu.make_async_copy(t_rows, t_out, osem.at[0]).wait()


def kernel(node_type, edge_index, table):
    mesh = plsc.VectorSubcoreMesh(core_axis_name="c", subcore_axis_name="s")
    k = pl.kernel(
        _sc_body,
        mesh=mesh,
        out_type=jax.ShapeDtypeStruct((N_EDGES, DIM), jnp.float32),
        compiler_params=pltpu.CompilerParams(needs_layout_passes=False),
        scratch_types=[
            pltpu.VMEM((N_NODES,), jnp.int32),
            pltpu.VMEM((2, EIW), jnp.int32),
            pltpu.VMEM((E_PER_W,), jnp.int32),
            pltpu.VMEM((NBUF, CHUNK, DIM), jnp.float32),
            pltpu.VMEM_SHARED((EDGE_NUM, DIM), jnp.float32),
            pltpu.SemaphoreType.DMA,
            pltpu.SemaphoreType.DMA((NBUF,)),
            pltpu.SemaphoreType.DMA((NBUF,)),
        ],
    )
    return k(node_type, edge_index, table)
